# baseline (speedup 1.0000x reference)
import functools

import jax
import jax.numpy as jnp
from jax import lax
from jax.experimental import pallas as pl
from jax.experimental.pallas import tpu as pltpu
from jax.experimental.pallas import tpu_sc as plsc

_D = 64
_NC = 2
_NS = 16
_NW = _NC * _NS
_V = 1_000_000
_VCOLS = (_V + 127) // 128
_PROWS = _VCOLS * 64
_B = 4096
_S = 200
_WCOLS = _B // 128
_QTILES = _S // 8

_mesh = functools.partial(
    plsc.VectorSubcoreMesh,
    core_axis_name="c", subcore_axis_name="s",
    num_cores=_NC, num_subcores=_NS,
)


def _widx():
    return lax.axis_index("s") * _NC + lax.axis_index("c")


@functools.cache
def _transpose_call():
    @functools.partial(
        pl.kernel,
        out_type=jax.ShapeDtypeStruct((_PROWS, 128), jnp.float32),
        mesh=_mesh(),
        scratch_types=[
            pltpu.VMEM((64, 137), jnp.float32),
            pltpu.VMEM((64, 128), jnp.float32),
        ],
        compiler_params=pltpu.CompilerParams(use_tc_tiling_on_sc=True, needs_layout_passes=False),
    )
    def body(w_hbm, scratch_hbm, wbuf, pbuf):
        wid = _widx()
        iota = lax.iota(jnp.int32, 16)
        zero16 = jnp.full((16,), 0, jnp.int32)
        n_c = 244 + jnp.where(wid < _VCOLS - 244 * _NW, 1, 0)

        def col(j, carry):
            c = wid + j * _NW
            pltpu.sync_copy(w_hbm.at[:, pl.ds(c * 128, 128)],
                            wbuf.at[:, pl.ds(0, 128)])

            def qloop(q, qcarry):
                for h in range(2):
                    colv = zero16 + (2 * q + h)
                    for dg in range(4):
                        v = plsc.load_gather(wbuf, [iota + 16 * dg, colv])
                        pbuf[q, pl.ds(64 * h + 16 * dg, 16)] = v
                return qcarry

            lax.fori_loop(0, 64, qloop, 0)
            pltpu.sync_copy(pbuf, scratch_hbm.at[pl.ds(c * 64, 64), :])
            return carry

        lax.fori_loop(0, n_c, col, 0)

    return body


@functools.cache
def _gather_call():
    @functools.partial(
        pl.kernel,
        out_type=jax.ShapeDtypeStruct((_S, 8, _WCOLS, 8, 128), jnp.float32),
        mesh=_mesh(),
        scratch_types=[
            pltpu.VMEM((8, 128), jnp.int32),
            pltpu.VMEM((128,), jnp.int32),
            pltpu.VMEM((128,), jnp.int32),
            pltpu.VMEM((128, 137), jnp.float32),
            pltpu.VMEM((64, 128), jnp.float32),
            pltpu.SemaphoreType.DMA,
        ],
        compiler_params=pltpu.CompilerParams(use_tc_tiling_on_sc=True, needs_layout_passes=False),
    )
    def body(idx_hbm, scratch_hbm, out_hbm, ibuf, pidx, hcol, G, obuf, sem):
        wid = _widx()
        iota = lax.iota(jnp.int32, 16)

        def qloop(q, qcarry):
            pltpu.sync_copy(
                idx_hbm.at[pl.ds(8 * q, 8), pl.ds(128 * wid, 128)], ibuf)
            for r in range(8):
                for g in range(8):
                    v = ibuf[r, pl.ds(16 * g, 16)]
                    pidx[pl.ds(16 * g, 16)] = lax.shift_right_logical(v, 1)
                    hcol[pl.ds(16 * g, 16)] = (v & 1) * 64
                pltpu.async_copy(scratch_hbm.at[pidx],
                                 G.at[:, pl.ds(0, 128)], sem).wait()

                def kloop(kk, kcarry):
                    for g in range(8):
                        lanes = iota + 16 * g
                        cols = hcol[pl.ds(16 * g, 16)] + kk
                        v = plsc.load_gather(G, [lanes, cols])
                        obuf[kk, pl.ds(16 * g, 16)] = v
                    return kcarry

                lax.fori_loop(0, 64, kloop, 0)
                s = 8 * q + r
                for k in range(8):
                    pltpu.sync_copy(
                        obuf.at[pl.ds(8 * k, 8), :],
                        out_hbm.at[s, k, wid])
            return qcarry

        lax.fori_loop(0, _QTILES, qloop, 0)

    return body


def kernel(token_ids, weight):
    scratch = _transpose_call()(weight.T)
    out5 = _gather_call()(token_ids.T, scratch)
    return out5.transpose(2, 4, 0, 1, 3).reshape(_B, _S, _D)


# speedup vs baseline: 3.7664x; 3.7664x over previous
import functools

import jax
import jax.numpy as jnp
from jax import lax
from jax.experimental import pallas as pl
from jax.experimental.pallas import tpu as pltpu
from jax.experimental.pallas import tpu_sc as plsc

_D = 64
_NC = 2
_NS = 16
_NW = _NC * _NS
_NBUF = 2


@functools.cache
def _gather_call(n_total: int, chunk: int):
    b_per_w = n_total // _NW
    n_chunks = b_per_w // chunk
    assert n_chunks % _NBUF == 0 and n_chunks >= 2 * _NBUF
    mesh = plsc.VectorSubcoreMesh(
        core_axis_name="c", subcore_axis_name="s",
        num_cores=_NC, num_subcores=_NS,
    )

    @functools.partial(
        pl.kernel,
        out_type=jax.ShapeDtypeStruct((n_total, _D), jnp.float32),
        mesh=mesh,
        scratch_types=[
            pltpu.VMEM((b_per_w,), jnp.int32),
            [pltpu.VMEM((chunk, _D), jnp.float32) for _ in range(_NBUF)],
            [pltpu.SemaphoreType.DMA for _ in range(_NBUF)],
            [pltpu.SemaphoreType.DMA for _ in range(_NBUF)],
        ],
        compiler_params=pltpu.CompilerParams(use_tc_tiling_on_sc=False),
    )
    def body(idx_hbm, table_hbm, out_hbm, idx_v, rows, gsem, wsem):
        wid = lax.axis_index("s") * _NC + lax.axis_index("c")
        base0 = wid * b_per_w
        pltpu.sync_copy(idx_hbm.at[pl.ds(base0, b_per_w)], idx_v)

        def start_gather(i, b):
            pltpu.async_copy(
                table_hbm.at[idx_v.at[pl.ds(i * chunk, chunk)]],
                rows[b], gsem[b])

        for b in range(_NBUF):
            start_gather(b, b)

        def outer(j, carry):
            for b in range(_NBUF):
                i = j * _NBUF + b
                pltpu.make_async_copy(
                    out_hbm.at[pl.ds(0, chunk)], rows[b], gsem[b]).wait()
                out_slice = out_hbm.at[pl.ds(base0 + i * chunk, chunk)]
                pltpu.async_copy(rows[b], out_slice, wsem[b])
                pltpu.make_async_copy(rows[b], out_slice, wsem[b]).wait()

                @pl.when(i + _NBUF < n_chunks)
                def _():
                    start_gather(i + _NBUF, b)
            return carry

        lax.fori_loop(0, n_chunks // _NBUF, outer, 0)

    return body


def kernel(token_ids, weight):
    b, s = token_ids.shape
    flat = token_ids.T.reshape(b * s)
    out = _gather_call(b * s, 800)(flat, weight)
    return out.reshape(s, b, _D).transpose(1, 0, 2)
